# baseline (speedup 1.0000x reference)
import jax
import jax.numpy as jnp
from jax import lax
from jax.experimental import pallas as pl
from jax.experimental.pallas import tpu as pltpu
from jax.experimental.pallas import tpu_sc as plsc

NUM_CORES = 2
NUM_SUBCORES = 16
NUM_WORKERS = NUM_CORES * NUM_SUBCORES
BATCH = 16384
NROWS = 1000000
EMBED_DIM = 32
LANES = 16
PACK = 128 // EMBED_DIM
NLINES = NROWS // PACK
B_PER_W = BATCH // NUM_WORKERS
CHUNK = 128
N_CHUNKS = B_PER_W // CHUNK
GROUPS = CHUNK // LANES


def _mf_body(uid_hbm, iid_hbm, ut_hbm, it_hbm, ub_hbm, ib_hbm, gb_hbm,
             out_hbm,
             uids_v, iids_v, ulines_v, ilines_v, uidx_v, iidx_v,
             ub_v, ib_v, gb_v, out_v, bsem, gsem):
    wid = lax.axis_index("s") * NUM_CORES + lax.axis_index("c")
    base = wid * B_PER_W

    pltpu.sync_copy(uid_hbm.at[pl.ds(wid * N_CHUNKS, N_CHUNKS)], uids_v)
    pltpu.sync_copy(iid_hbm.at[pl.ds(wid * N_CHUNKS, N_CHUNKS)], iids_v)
    pltpu.sync_copy(gb_hbm, gb_v)

    bias_copies = []
    for q in range(N_CHUNKS):
        sl = pl.ds(q * CHUNK, CHUNK)
        bias_copies.append(
            pltpu.async_copy(ub_hbm.at[uids_v.at[q]], ub_v.at[sl], bsem))
        bias_copies.append(
            pltpu.async_copy(ib_hbm.at[iids_v.at[q]], ib_v.at[sl], bsem))

    def fire(c):
        buf = c % 2
        for g in range(GROUPS):
            ssl = pl.ds(g * LANES, LANES)
            uidx_v[buf, ssl] = uids_v[c, ssl] >> 2
            iidx_v[buf, ssl] = iids_v[c, ssl] >> 2
        pltpu.async_copy(ut_hbm.at[uidx_v.at[buf]], ulines_v.at[buf],
                         gsem.at[buf])
        pltpu.async_copy(it_hbm.at[iidx_v.at[buf]], ilines_v.at[buf],
                         gsem.at[buf])

    def drain(c):
        buf = c % 2
        pltpu.make_async_copy(ut_hbm.at[uidx_v.at[buf]], ulines_v.at[buf],
                              gsem.at[buf]).wait()
        pltpu.make_async_copy(it_hbm.at[iidx_v.at[buf]], ilines_v.at[buf],
                              gsem.at[buf]).wait()

    fire(0)
    lane = lax.iota(jnp.int32, LANES)

    for c in range(N_CHUNKS):
        if c + 1 < N_CHUNKS:
            fire(c + 1)
        drain(c)
        if c == 0:
            for cp in bias_copies:
                cp.wait()
        buf = c % 2

        def group_body(g, carry):
            rvu = uids_v[c, pl.ds(g * LANES, LANES)]
            rvi = iids_v[c, pl.ds(g * LANES, LANES)]
            qqu = (rvu & 3) * EMBED_DIM
            qqi = (rvi & 3) * EMBED_DIM
            acc = gb_v[...]
            for j in range(LANES):
                row = g * LANES + j
                qu = qqu[j]
                qi = qqi[j]
                u0 = ulines_v[buf, row, pl.ds(qu, LANES)]
                u1 = ulines_v[buf, row, pl.ds(qu + LANES, LANES)]
                i0 = ilines_v[buf, row, pl.ds(qi, LANES)]
                i1 = ilines_v[buf, row, pl.ds(qi + LANES, LANES)]
                dot = jnp.sum(u0 * i0 + u1 * i1)
                acc = jnp.where(lane == j, dot, acc)
            osl = pl.ds(c * CHUNK + g * LANES, LANES)
            out_v[osl] = acc + ub_v[osl] + ib_v[osl]
            return carry

        lax.fori_loop(0, GROUPS, group_body, 0)

    pltpu.sync_copy(out_v, out_hbm.at[pl.ds(base, B_PER_W)])


@jax.jit
def kernel(user_ids, item_ids, user_table, item_table, user_bias, item_bias,
           global_bias):
    uid2 = user_ids.astype(jnp.int32).reshape(BATCH // CHUNK, CHUNK)
    iid2 = item_ids.astype(jnp.int32).reshape(BATCH // CHUNK, CHUNK)
    ut4 = user_table.reshape(NLINES, 128)
    it4 = item_table.reshape(NLINES, 128)
    ub_flat = user_bias.reshape(-1)
    ib_flat = item_bias.reshape(-1)
    gb = jnp.broadcast_to(global_bias.reshape(1), (LANES,))

    mesh = plsc.VectorSubcoreMesh(
        core_axis_name="c", subcore_axis_name="s",
        num_cores=NUM_CORES, num_subcores=NUM_SUBCORES)

    run = pl.kernel(
        _mf_body,
        out_type=jax.ShapeDtypeStruct((BATCH,), jnp.float32),
        mesh=mesh,
        compiler_params=pltpu.CompilerParams(
            needs_layout_passes=False, use_tc_tiling_on_sc=False),
        scratch_types=[
            pltpu.VMEM((N_CHUNKS, CHUNK), jnp.int32),
            pltpu.VMEM((N_CHUNKS, CHUNK), jnp.int32),
            pltpu.VMEM((2, CHUNK, 128), jnp.float32),
            pltpu.VMEM((2, CHUNK, 128), jnp.float32),
            pltpu.VMEM((2, CHUNK), jnp.int32),
            pltpu.VMEM((2, CHUNK), jnp.int32),
            pltpu.VMEM((B_PER_W,), jnp.float32),
            pltpu.VMEM((B_PER_W,), jnp.float32),
            pltpu.VMEM((LANES,), jnp.float32),
            pltpu.VMEM((B_PER_W,), jnp.float32),
            pltpu.SemaphoreType.DMA,
            pltpu.SemaphoreType.DMA((2,)),
        ],
    )
    return run(uid2, iid2, ut4, it4, ub_flat, ib_flat, gb)
